# baseline (speedup 1.0000x reference)
import functools

import jax
import jax.numpy as jnp
from jax import lax
from jax.experimental import pallas as pl
from jax.experimental.pallas import tpu as pltpu
from jax.experimental.pallas import tpu_sc as plsc

_NEG = -1e9

_N = 10000
_E = 160000
_D = 256
_H = 512

_NC = 2
_NS = 16
_NW = _NC * _NS


def _mm_body(nf_ref, w1a_ref, w1b_ref, b1_ref, a_ref, b_ref):
    x = nf_ref[...]
    a_ref[...] = (
        jnp.dot(x, w1a_ref[...], preferred_element_type=jnp.float32) + b1_ref[...]
    )
    b_ref[...] = jnp.dot(x, w1b_ref[...], preferred_element_type=jnp.float32)


def _node_tables(nf, w1a, w1b, b1):
    bn = 400
    return pl.pallas_call(
        _mm_body,
        grid=(_N // bn,),
        in_specs=[
            pl.BlockSpec((bn, _D), lambda i: (i, 0)),
            pl.BlockSpec((_D, _H), lambda i: (0, 0)),
            pl.BlockSpec((_D, _H), lambda i: (0, 0)),
            pl.BlockSpec((1, _H), lambda i: (0, 0)),
        ],
        out_specs=[
            pl.BlockSpec((bn, _H), lambda i: (i, 0)),
            pl.BlockSpec((bn, _H), lambda i: (i, 0)),
        ],
        out_shape=[
            jax.ShapeDtypeStruct((_N, _H), jnp.float32),
            jax.ShapeDtypeStruct((_N, _H), jnp.float32),
        ],
    )(nf, w1a, w1b, b1)


_CH2 = 40


def _gather_add_body(a_hbm, b_hbm, dst_hbm, src_hbm, z_hbm, dv, sv, arows, brows,
                     sem_a, sem_b):
    epw = _E // _NW
    wid = lax.axis_index("s") * _NC + lax.axis_index("c")
    base0 = wid * epw

    def chunk(i, _):
        base = base0 + i * _CH2
        pltpu.sync_copy(dst_hbm.at[pl.ds(base, _CH2)], dv)
        pltpu.sync_copy(src_hbm.at[pl.ds(base, _CH2)], sv)
        ca = pltpu.async_copy(a_hbm.at[dv], arows, sem_a)
        cb = pltpu.async_copy(b_hbm.at[sv], brows, sem_b)
        ca.wait()
        cb.wait()

        def row(r, _):
            def col(c, _):
                sl = pl.ds(c * 16, 16)
                arows[r, sl] = arows[r, sl] + brows[r, sl]
                return 0

            return lax.fori_loop(0, _H // 16, col, 0)

        lax.fori_loop(0, _CH2, row, 0)
        pltpu.sync_copy(arows, z_hbm.at[pl.ds(base, _CH2)])
        return 0

    lax.fori_loop(0, epw // _CH2, chunk, 0)


def _gather_add(a_tab, b_tab, dst, src):
    mesh = plsc.VectorSubcoreMesh(core_axis_name="c", subcore_axis_name="s")
    return pl.kernel(
        _gather_add_body,
        out_type=jax.ShapeDtypeStruct((_E, _H), jnp.float32),
        mesh=mesh,
        scratch_types=[
            pltpu.VMEM((_CH2,), jnp.int32),
            pltpu.VMEM((_CH2,), jnp.int32),
            pltpu.VMEM((_CH2, _H), jnp.float32),
            pltpu.VMEM((_CH2, _H), jnp.float32),
            pltpu.SemaphoreType.DMA,
            pltpu.SemaphoreType.DMA,
        ],
    )(a_tab, b_tab, dst, src)


def _mish_dot_body(z_ref, w2_ref, b2_ref, val_ref):
    z = z_ref[...]
    sp = jnp.maximum(z, 0.0) + jnp.log1p(jnp.exp(-jnp.abs(z)))
    m = z * jnp.tanh(sp)
    val_ref[...] = jnp.sum(m * w2_ref[...], axis=1, keepdims=True) + b2_ref[0, 0]


def _mish_dot(z, w2_row, b2):
    be = 1000
    g = _E // be
    out = pl.pallas_call(
        _mish_dot_body,
        grid=(g,),
        in_specs=[
            pl.BlockSpec((be, _H), lambda i: (i, 0)),
            pl.BlockSpec((1, _H), lambda i: (0, 0)),
            pl.BlockSpec((1, 1), lambda i: (0, 0)),
        ],
        out_specs=pl.BlockSpec((be, 1), lambda i: (i, 0)),
        out_shape=jax.ShapeDtypeStruct((_E, 1), jnp.float32),
    )(z, w2_row, b2)
    return out.reshape(_E)


_CH4 = 80


def _make_mailbox(m):
    outsz = _N * m
    opw = outsz // _NW

    def body(valext_hbm, gidx_hbm, out_hbm, iv, vals, sem):
        wid = lax.axis_index("s") * _NC + lax.axis_index("c")
        base0 = wid * opw

        def chunk(i, _):
            base = base0 + i * _CH4
            pltpu.sync_copy(gidx_hbm.at[pl.ds(base, _CH4)], iv)
            pltpu.async_copy(valext_hbm.at[iv], vals, sem).wait()
            pltpu.sync_copy(vals, out_hbm.at[pl.ds(base, _CH4)])
            return 0

        lax.fori_loop(0, opw // _CH4, chunk, 0)

    mesh = plsc.VectorSubcoreMesh(core_axis_name="c", subcore_axis_name="s")
    return pl.kernel(
        body,
        out_type=jax.ShapeDtypeStruct((outsz,), jnp.float32),
        mesh=mesh,
        scratch_types=[
            pltpu.VMEM((_CH4,), jnp.int32),
            pltpu.VMEM((_CH4,), jnp.float32),
            pltpu.SemaphoreType.DMA,
        ],
    )


_M = 64


def kernel(node_feature, W1, b1, W2, b2, src_idx, dst_idx, maximum_num_enemy,
           attack_edge_type_index):
    m = _M
    nf = node_feature.astype(jnp.float32)
    dst = dst_idx.astype(jnp.int32)
    src = src_idx.astype(jnp.int32)

    w1a = W1[:_D]
    w1b = W1[_D:]
    b1r = b1.reshape(1, _H)
    w2_row = W2.reshape(1, _H)
    b2r = b2.reshape(1, 1)

    a_tab, b_tab = _node_tables(nf, w1a, w1b, b1r)
    z = _gather_add(a_tab, b_tab, dst, src)
    val = _mish_dot(z, w2_row, b2r)

    val_ext = jnp.concatenate([val, jnp.full((64,), _NEG, jnp.float32)])

    counts = jnp.bincount(dst, length=_N)
    starts = jnp.cumsum(counts) - counts
    j = jnp.arange(m, dtype=jnp.int32)
    keep = j[None, :] < jnp.minimum(counts, maximum_num_enemy)[:, None]
    gidx = jnp.where(keep, starts[:, None].astype(jnp.int32) + j[None, :], _E)
    gidx = gidx.reshape(-1).astype(jnp.int32)

    out_flat = _make_mailbox(m)(val_ext, gidx)
    return out_flat.reshape(_N, m)


# speedup vs baseline: 1.2420x; 1.2420x over previous
import jax
import jax.numpy as jnp
from jax import lax
from jax.experimental import pallas as pl
from jax.experimental.pallas import tpu as pltpu
from jax.experimental.pallas import tpu_sc as plsc

_NEG = -1e9

_N = 10000
_E = 160000
_D = 256
_H = 512
_M = 64

_NC = 2
_NS = 16
_NW = _NC * _NS


def _mm_body(nf_ref, w1a_ref, w1b_ref, b1_ref, a_ref, b_ref):
    x = nf_ref[...]
    a_ref[...] = (
        jnp.dot(x, w1a_ref[...], preferred_element_type=jnp.float32) + b1_ref[...]
    )
    b_ref[...] = jnp.dot(x, w1b_ref[...], preferred_element_type=jnp.float32)


def _node_tables(nf, w1a, w1b, b1):
    bn = 400
    return pl.pallas_call(
        _mm_body,
        grid=(_N // bn,),
        in_specs=[
            pl.BlockSpec((bn, _D), lambda i: (i, 0)),
            pl.BlockSpec((_D, _H), lambda i: (0, 0)),
            pl.BlockSpec((_D, _H), lambda i: (0, 0)),
            pl.BlockSpec((1, _H), lambda i: (0, 0)),
        ],
        out_specs=[
            pl.BlockSpec((bn, _H), lambda i: (i, 0)),
            pl.BlockSpec((bn, _H), lambda i: (i, 0)),
        ],
        out_shape=[
            jax.ShapeDtypeStruct((_N, _H), jnp.float32),
            jax.ShapeDtypeStruct((_N, _H), jnp.float32),
        ],
    )(nf, w1a, w1b, b1)


_CH2 = 40


def _gather_add_body(a_hbm, b_hbm, dst_hbm, src_hbm, z_hbm,
                     dv0, sv0, ar0, br0, zb0,
                     dv1, sv1, ar1, br1, zb1,
                     sem_g0, sem_g1, sem_s0, sem_s1):
    epw = _E // _NW
    nchunk = epw // _CH2
    wid = lax.axis_index("s") * _NC + lax.axis_index("c")
    base0 = wid * epw
    bufs = ((dv0, sv0, ar0, br0, zb0, sem_g0, sem_s0),
            (dv1, sv1, ar1, br1, zb1, sem_g1, sem_s1))

    def fetch(k, b):
        dv, sv, ar, br, zb, sg, ss = bufs[b]
        base = base0 + k * _CH2
        pltpu.sync_copy(dst_hbm.at[pl.ds(base, _CH2)], dv)
        pltpu.sync_copy(src_hbm.at[pl.ds(base, _CH2)], sv)
        pltpu.async_copy(a_hbm.at[dv], ar, sg)
        pltpu.async_copy(b_hbm.at[sv], br, sg)

    def process(k, b):
        dv, sv, ar, br, zb, sg, ss = bufs[b]
        pltpu.make_async_copy(a_hbm.at[dv], ar, sg).wait()
        pltpu.make_async_copy(b_hbm.at[sv], br, sg).wait()

        def row(r, _):
            for c in range(_H // 16):
                sl = pl.ds(c * 16, 16)
                zb[r, sl] = ar[r, sl] + br[r, sl]
            return 0

        lax.fori_loop(0, _CH2, row, 0)

        @pl.when(k >= 2)
        def _():
            pltpu.make_async_copy(zb, z_hbm.at[pl.ds(base0, _CH2)], ss).wait()

        pltpu.async_copy(zb, z_hbm.at[pl.ds(base0 + k * _CH2, _CH2)], ss)

        @pl.when(k + 2 < nchunk)
        def _():
            fetch(k + 2, b)

    fetch(0, 0)
    fetch(1, 1)

    def pair(g, _):
        k0 = g * 2
        process(k0, 0)
        process(k0 + 1, 1)
        return 0

    lax.fori_loop(0, nchunk // 2, pair, 0)
    if nchunk % 2 == 1:
        process(nchunk - 1, 0)

    pltpu.make_async_copy(zb0, z_hbm.at[pl.ds(base0, _CH2)], sem_s0).wait()
    pltpu.make_async_copy(zb1, z_hbm.at[pl.ds(base0, _CH2)], sem_s1).wait()


def _gather_add(a_tab, b_tab, dst, src):
    mesh = plsc.VectorSubcoreMesh(core_axis_name="c", subcore_axis_name="s")
    buf = lambda: [
        pltpu.VMEM((_CH2,), jnp.int32),
        pltpu.VMEM((_CH2,), jnp.int32),
        pltpu.VMEM((_CH2, _H), jnp.float32),
        pltpu.VMEM((_CH2, _H), jnp.float32),
        pltpu.VMEM((_CH2, _H), jnp.float32),
    ]
    return pl.kernel(
        _gather_add_body,
        out_type=jax.ShapeDtypeStruct((_E, _H), jnp.float32),
        mesh=mesh,
        scratch_types=buf() + buf() + [
            pltpu.SemaphoreType.DMA,
            pltpu.SemaphoreType.DMA,
            pltpu.SemaphoreType.DMA,
            pltpu.SemaphoreType.DMA,
        ],
    )(a_tab, b_tab, dst, src)


def _mish_dot_body(z_ref, w2_ref, b2_ref, val_ref):
    z = z_ref[...]
    sp = jnp.maximum(z, 0.0) + jnp.log1p(jnp.exp(-jnp.abs(z)))
    m = z * jnp.tanh(sp)
    val_ref[...] = jnp.sum(m * w2_ref[...], axis=1, keepdims=True) + b2_ref[0, 0]


def _mish_dot(z, w2_row, b2):
    be = 1000
    g = _E // be
    out = pl.pallas_call(
        _mish_dot_body,
        grid=(g,),
        in_specs=[
            pl.BlockSpec((be, _H), lambda i: (i, 0)),
            pl.BlockSpec((1, _H), lambda i: (0, 0)),
            pl.BlockSpec((1, 1), lambda i: (0, 0)),
        ],
        out_specs=pl.BlockSpec((be, 1), lambda i: (i, 0)),
        out_shape=jax.ShapeDtypeStruct((_E, 1), jnp.float32),
    )(z, w2_row, b2)
    return out.reshape(_E)


_CW = 100
_SR = 8


def _mailbox_body(valext_hbm, gidx_hbm, out_hbm,
                  iv0, vb0, iv1, vb1, sem_g0, sem_g1, sem_s0, sem_s1):
    nrow = (_N * _M) // _CW
    rpw = nrow // _NW
    nsuper = rpw // _SR
    wid = lax.axis_index("s") * _NC + lax.axis_index("c")
    r0w = wid * rpw
    bufs = ((iv0, vb0, sem_g0, sem_s0), (iv1, vb1, sem_g1, sem_s1))

    def fetch(s, b):
        iv, vb, sg, ss = bufs[b]
        pltpu.sync_copy(gidx_hbm.at[pl.ds(r0w + s * _SR, _SR)], iv)
        for j in range(_SR):
            pltpu.async_copy(valext_hbm.at[iv.at[j]], vb.at[j], sg)

    def process(s, b):
        iv, vb, sg, ss = bufs[b]
        for j in range(_SR):
            pltpu.make_async_copy(valext_hbm.at[iv.at[j]], vb.at[j], sg).wait()

        @pl.when(s >= 2)
        def _():
            pltpu.make_async_copy(vb, out_hbm.at[pl.ds(r0w, _SR)], ss).wait()

        pltpu.async_copy(vb, out_hbm.at[pl.ds(r0w + s * _SR, _SR)], ss)

        @pl.when(s + 2 < nsuper)
        def _():
            fetch(s + 2, b)

    fetch(0, 0)
    fetch(1, 1)

    def pair(g, _):
        s0 = g * 2
        process(s0, 0)
        process(s0 + 1, 1)
        return 0

    lax.fori_loop(0, nsuper // 2, pair, 0)
    if nsuper % 2 == 1:
        process(nsuper - 1, 0)

    pltpu.make_async_copy(vb0, out_hbm.at[pl.ds(r0w, _SR)], sem_s0).wait()
    pltpu.make_async_copy(vb1, out_hbm.at[pl.ds(r0w, _SR)], sem_s1).wait()


def _mailbox(val_ext, gidx2d):
    nrow = (_N * _M) // _CW
    mesh = plsc.VectorSubcoreMesh(core_axis_name="c", subcore_axis_name="s")
    return pl.kernel(
        _mailbox_body,
        out_type=jax.ShapeDtypeStruct((nrow, _CW), jnp.float32),
        mesh=mesh,
        scratch_types=[
            pltpu.VMEM((_SR, _CW), jnp.int32),
            pltpu.VMEM((_SR, _CW), jnp.float32),
            pltpu.VMEM((_SR, _CW), jnp.int32),
            pltpu.VMEM((_SR, _CW), jnp.float32),
            pltpu.SemaphoreType.DMA,
            pltpu.SemaphoreType.DMA,
            pltpu.SemaphoreType.DMA,
            pltpu.SemaphoreType.DMA,
        ],
    )(val_ext, gidx2d)


def kernel(node_feature, W1, b1, W2, b2, src_idx, dst_idx, maximum_num_enemy,
           attack_edge_type_index):
    m = _M
    nf = node_feature.astype(jnp.float32)
    dst = dst_idx.astype(jnp.int32)
    src = src_idx.astype(jnp.int32)

    w1a = W1[:_D]
    w1b = W1[_D:]
    b1r = b1.reshape(1, _H)
    w2_row = W2.reshape(1, _H)
    b2r = b2.reshape(1, 1)

    a_tab, b_tab = _node_tables(nf, w1a, w1b, b1r)
    z = _gather_add(a_tab, b_tab, dst, src)
    val = _mish_dot(z, w2_row, b2r)

    val_ext = jnp.concatenate([val, jnp.full((64,), _NEG, jnp.float32)])

    counts = jnp.bincount(dst, length=_N)
    starts = jnp.cumsum(counts) - counts
    j = jnp.arange(m, dtype=jnp.int32)
    keep = j[None, :] < jnp.minimum(counts, maximum_num_enemy)[:, None]
    gidx = jnp.where(keep, starts[:, None].astype(jnp.int32) + j[None, :], _E)
    gidx2d = gidx.reshape((_N * _M) // _CW, _CW).astype(jnp.int32)

    out2d = _mailbox(val_ext, gidx2d)
    return out2d.reshape(_N, m)


# speedup vs baseline: 3.8177x; 3.0738x over previous
import jax
import jax.numpy as jnp
from jax import lax
from jax.experimental import pallas as pl
from jax.experimental.pallas import tpu as pltpu
from jax.experimental.pallas import tpu_sc as plsc

_NEG = -1e9

_N = 10000
_E = 160000
_D = 256
_H = 512
_M = 64

_NC = 2
_NS = 16
_NW = _NC * _NS


def _mm_body(nf_ref, w1a_ref, w1b_ref, b1_ref, a_ref, b_ref):
    x = nf_ref[...]
    a_ref[...] = (
        jnp.dot(x, w1a_ref[...], preferred_element_type=jnp.float32) + b1_ref[...]
    )
    b_ref[...] = jnp.dot(x, w1b_ref[...], preferred_element_type=jnp.float32)


def _node_tables(nf, w1a, w1b, b1):
    bn = 400
    return pl.pallas_call(
        _mm_body,
        grid=(_N // bn,),
        in_specs=[
            pl.BlockSpec((bn, _D), lambda i: (i, 0)),
            pl.BlockSpec((_D, _H), lambda i: (0, 0)),
            pl.BlockSpec((_D, _H), lambda i: (0, 0)),
            pl.BlockSpec((1, _H), lambda i: (0, 0)),
        ],
        out_specs=[
            pl.BlockSpec((bn, _H), lambda i: (i, 0)),
            pl.BlockSpec((bn, _H), lambda i: (i, 0)),
        ],
        out_shape=[
            jax.ShapeDtypeStruct((_N, _H), jnp.float32),
            jax.ShapeDtypeStruct((_N, _H), jnp.float32),
        ],
    )(nf, w1a, w1b, b1)


_CH2 = 40


def _gather_add_body(a_hbm, b_hbm, dst_hbm, src_hbm, z_hbm,
                     dv0, sv0, ar0, br0, zb0,
                     dv1, sv1, ar1, br1, zb1,
                     sem_g0, sem_g1, sem_s0, sem_s1):
    epw = _E // _NW
    nchunk = epw // _CH2
    wid = lax.axis_index("s") * _NC + lax.axis_index("c")
    base0 = wid * epw
    bufs = ((dv0, sv0, ar0, br0, zb0, sem_g0, sem_s0),
            (dv1, sv1, ar1, br1, zb1, sem_g1, sem_s1))

    def fetch(k, b):
        dv, sv, ar, br, zb, sg, ss = bufs[b]
        base = base0 + k * _CH2
        pltpu.sync_copy(dst_hbm.at[pl.ds(base, _CH2)], dv)
        pltpu.sync_copy(src_hbm.at[pl.ds(base, _CH2)], sv)
        pltpu.async_copy(a_hbm.at[dv], ar, sg)
        pltpu.async_copy(b_hbm.at[sv], br, sg)

    def process(k, b):
        dv, sv, ar, br, zb, sg, ss = bufs[b]
        pltpu.make_async_copy(a_hbm.at[dv], ar, sg).wait()
        pltpu.make_async_copy(b_hbm.at[sv], br, sg).wait()

        def row(r, _):
            for c in range(_H // 16):
                sl = pl.ds(c * 16, 16)
                zb[r, sl] = ar[r, sl] + br[r, sl]
            return 0

        lax.fori_loop(0, _CH2, row, 0)

        @pl.when(k >= 2)
        def _():
            pltpu.make_async_copy(zb, z_hbm.at[pl.ds(base0, _CH2)], ss).wait()

        pltpu.async_copy(zb, z_hbm.at[pl.ds(base0 + k * _CH2, _CH2)], ss)

        @pl.when(k + 2 < nchunk)
        def _():
            fetch(k + 2, b)

    fetch(0, 0)
    fetch(1, 1)

    def pair(g, _):
        k0 = g * 2
        process(k0, 0)
        process(k0 + 1, 1)
        return 0

    lax.fori_loop(0, nchunk // 2, pair, 0)
    if nchunk % 2 == 1:
        process(nchunk - 1, 0)

    pltpu.make_async_copy(zb0, z_hbm.at[pl.ds(base0, _CH2)], sem_s0).wait()
    pltpu.make_async_copy(zb1, z_hbm.at[pl.ds(base0, _CH2)], sem_s1).wait()


def _gather_add(a_tab, b_tab, dst, src):
    mesh = plsc.VectorSubcoreMesh(core_axis_name="c", subcore_axis_name="s")
    buf = lambda: [
        pltpu.VMEM((_CH2,), jnp.int32),
        pltpu.VMEM((_CH2,), jnp.int32),
        pltpu.VMEM((_CH2, _H), jnp.float32),
        pltpu.VMEM((_CH2, _H), jnp.float32),
        pltpu.VMEM((_CH2, _H), jnp.float32),
    ]
    return pl.kernel(
        _gather_add_body,
        out_type=jax.ShapeDtypeStruct((_E, _H), jnp.float32),
        mesh=mesh,
        scratch_types=buf() + buf() + [
            pltpu.SemaphoreType.DMA,
            pltpu.SemaphoreType.DMA,
            pltpu.SemaphoreType.DMA,
            pltpu.SemaphoreType.DMA,
        ],
    )(a_tab, b_tab, dst, src)


def _mish_dot_body(z_ref, w2_ref, b2_ref, val_ref):
    z = z_ref[...]
    sp = jnp.maximum(z, 0.0) + jnp.log1p(jnp.exp(-jnp.abs(z)))
    m = z * jnp.tanh(sp)
    val_ref[...] = jnp.sum(m * w2_ref[...], axis=1, keepdims=True) + b2_ref[0, 0]


def _mish_dot(z, w2_row, b2):
    be = 1000
    g = _E // be
    out = pl.pallas_call(
        _mish_dot_body,
        grid=(g,),
        in_specs=[
            pl.BlockSpec((be, _H), lambda i: (i, 0)),
            pl.BlockSpec((1, _H), lambda i: (0, 0)),
            pl.BlockSpec((1, 1), lambda i: (0, 0)),
        ],
        out_specs=pl.BlockSpec((be, 1), lambda i: (i, 0)),
        out_shape=jax.ShapeDtypeStruct((_E, 1), jnp.float32),
    )(z, w2_row, b2)
    return out.reshape(_E)


_RB = 80


def _mailbox_body(starts_ref, counts_ref, mne_ref, val_ref, out_ref):
    i = pl.program_id(0)
    iot = lax.broadcasted_iota(jnp.int32, (1, _M), 1)
    for r in range(_RB):
        n = i * _RB + r
        s = starts_ref[n]
        sa = pl.multiple_of((s // 128) * 128, 128)
        off = s - sa
        c = jnp.minimum(counts_ref[n], mne_ref[0])
        w = val_ref[pl.ds(0, 1), pl.ds(sa, 256)]
        w = pltpu.roll(w, 256 - off, 1)[:, :_M]
        out_ref[pl.ds(r, 1), :] = jnp.where(iot < c, w, _NEG)


def _mailbox(starts, counts, mne, val_row):
    return pl.pallas_call(
        _mailbox_body,
        grid=(_N // _RB,),
        in_specs=[
            pl.BlockSpec(memory_space=pltpu.SMEM),
            pl.BlockSpec(memory_space=pltpu.SMEM),
            pl.BlockSpec(memory_space=pltpu.SMEM),
            pl.BlockSpec((1, _E + 256), lambda i: (0, 0)),
        ],
        out_specs=pl.BlockSpec((_RB, _M), lambda i: (i, 0)),
        out_shape=jax.ShapeDtypeStruct((_N, _M), jnp.float32),
    )(starts, counts, mne, val_row)


def kernel(node_feature, W1, b1, W2, b2, src_idx, dst_idx, maximum_num_enemy,
           attack_edge_type_index):
    nf = node_feature.astype(jnp.float32)
    dst = dst_idx.astype(jnp.int32)
    src = src_idx.astype(jnp.int32)

    w1a = W1[:_D]
    w1b = W1[_D:]
    b1r = b1.reshape(1, _H)
    w2_row = W2.reshape(1, _H)
    b2r = b2.reshape(1, 1)

    a_tab, b_tab = _node_tables(nf, w1a, w1b, b1r)
    z = _gather_add(a_tab, b_tab, dst, src)
    val = _mish_dot(z, w2_row, b2r)

    counts = jnp.bincount(dst, length=_N).astype(jnp.int32)
    starts = (jnp.cumsum(counts) - counts).astype(jnp.int32)
    val_row = jnp.concatenate([val, jnp.zeros((256,), jnp.float32)]).reshape(1, _E + 256)
    mne = jnp.asarray(maximum_num_enemy, jnp.int32).reshape(1)

    return _mailbox(starts, counts, mne, val_row)


# speedup vs baseline: 4.2109x; 1.1030x over previous
import jax
import jax.numpy as jnp
from jax import lax
from jax.experimental import pallas as pl
from jax.experimental.pallas import tpu as pltpu
from jax.experimental.pallas import tpu_sc as plsc

_NEG = -1e9

_N = 10000
_E = 160000
_D = 256
_H = 512
_M = 64

_NC = 2
_NS = 16
_NW = _NC * _NS


def _mm_body(nf_ref, w1a_ref, w1b_ref, b1_ref, a_ref, b_ref):
    x = nf_ref[...]
    a_ref[...] = (
        jnp.dot(x, w1a_ref[...], preferred_element_type=jnp.float32) + b1_ref[...]
    )
    b_ref[...] = jnp.dot(x, w1b_ref[...], preferred_element_type=jnp.float32)


def _node_tables(nf, w1a, w1b, b1):
    bn = 400
    return pl.pallas_call(
        _mm_body,
        grid=(_N // bn,),
        in_specs=[
            pl.BlockSpec((bn, _D), lambda i: (i, 0)),
            pl.BlockSpec((_D, _H), lambda i: (0, 0)),
            pl.BlockSpec((_D, _H), lambda i: (0, 0)),
            pl.BlockSpec((1, _H), lambda i: (0, 0)),
        ],
        out_specs=[
            pl.BlockSpec((bn, _H), lambda i: (i, 0)),
            pl.BlockSpec((bn, _H), lambda i: (i, 0)),
        ],
        out_shape=[
            jax.ShapeDtypeStruct((_N, _H), jnp.float32),
            jax.ShapeDtypeStruct((_N, _H), jnp.float32),
        ],
    )(nf, w1a, w1b, b1)


_CH2 = 40


def _gather_add_body(a_hbm, b_hbm, dst_hbm, src_hbm, z_hbm,
                     dv0, sv0, ar0, br0, zb0,
                     dv1, sv1, ar1, br1, zb1,
                     sem_g0, sem_g1, sem_s0, sem_s1):
    eg = z_hbm.shape[0]
    epw = eg // _NW
    nchunk = epw // _CH2
    wid = lax.axis_index("s") * _NC + lax.axis_index("c")
    base0 = wid * epw
    bufs = ((dv0, sv0, ar0, br0, zb0, sem_g0, sem_s0),
            (dv1, sv1, ar1, br1, zb1, sem_g1, sem_s1))

    def fetch(k, b):
        dv, sv, ar, br, zb, sg, ss = bufs[b]
        base = base0 + k * _CH2
        pltpu.sync_copy(dst_hbm.at[pl.ds(base, _CH2)], dv)
        pltpu.sync_copy(src_hbm.at[pl.ds(base, _CH2)], sv)
        pltpu.async_copy(a_hbm.at[dv], ar, sg)
        pltpu.async_copy(b_hbm.at[sv], br, sg)

    def process(k, b):
        dv, sv, ar, br, zb, sg, ss = bufs[b]
        pltpu.make_async_copy(a_hbm.at[dv], ar, sg).wait()
        pltpu.make_async_copy(b_hbm.at[sv], br, sg).wait()

        def row(r, _):
            for c in range(_H // 16):
                sl = pl.ds(c * 16, 16)
                zb[r, sl] = ar[r, sl] + br[r, sl]
            return 0

        lax.fori_loop(0, _CH2, row, 0)

        @pl.when(k >= 2)
        def _():
            pltpu.make_async_copy(zb, z_hbm.at[pl.ds(base0, _CH2)], ss).wait()

        pltpu.async_copy(zb, z_hbm.at[pl.ds(base0 + k * _CH2, _CH2)], ss)

        @pl.when(k + 2 < nchunk)
        def _():
            fetch(k + 2, b)

    fetch(0, 0)
    fetch(1, 1)

    def pair(g, _):
        k0 = g * 2
        process(k0, 0)
        process(k0 + 1, 1)
        return 0

    lax.fori_loop(0, nchunk // 2, pair, 0)
    if nchunk % 2 == 1:
        process(nchunk - 1, 0)

    pltpu.make_async_copy(zb0, z_hbm.at[pl.ds(base0, _CH2)], sem_s0).wait()
    pltpu.make_async_copy(zb1, z_hbm.at[pl.ds(base0, _CH2)], sem_s1).wait()


def _gather_add(a_tab, b_tab, dst, src):
    mesh = plsc.VectorSubcoreMesh(core_axis_name="c", subcore_axis_name="s")
    buf = lambda: [
        pltpu.VMEM((_CH2,), jnp.int32),
        pltpu.VMEM((_CH2,), jnp.int32),
        pltpu.VMEM((_CH2, _H), jnp.float32),
        pltpu.VMEM((_CH2, _H), jnp.float32),
        pltpu.VMEM((_CH2, _H), jnp.float32),
    ]
    return pl.kernel(
        _gather_add_body,
        out_type=jax.ShapeDtypeStruct((dst.shape[0], _H), jnp.float32),
        mesh=mesh,
        scratch_types=buf() + buf() + [
            pltpu.SemaphoreType.DMA,
            pltpu.SemaphoreType.DMA,
            pltpu.SemaphoreType.DMA,
            pltpu.SemaphoreType.DMA,
        ],
    )(a_tab, b_tab, dst, src)


def _mish_dot_body(z_ref, w2_ref, b2_ref, val_ref):
    z = z_ref[...]
    sp = jnp.maximum(z, 0.0) + jnp.log1p(jnp.exp(-jnp.abs(z)))
    m = z * jnp.tanh(sp)
    val_ref[...] = jnp.sum(m * w2_ref[...], axis=1, keepdims=True) + b2_ref[0, 0]


def _mish_dot(z, w2_row, b2):
    be = 1600
    eg = z.shape[0]
    g = eg // be
    out = pl.pallas_call(
        _mish_dot_body,
        grid=(g,),
        in_specs=[
            pl.BlockSpec((be, _H), lambda i: (i, 0)),
            pl.BlockSpec((1, _H), lambda i: (0, 0)),
            pl.BlockSpec((1, 1), lambda i: (0, 0)),
        ],
        out_specs=pl.BlockSpec((be, 1), lambda i: (i, 0)),
        out_shape=jax.ShapeDtypeStruct((eg, 1), jnp.float32),
    )(z, w2_row, b2)
    return out.reshape(eg)


_RB = 80


def _mailbox_body(starts_ref, counts_ref, mne_ref, val_ref, out_ref):
    i = pl.program_id(0)
    iot = lax.broadcasted_iota(jnp.int32, (1, _M), 1)
    for r in range(_RB):
        n = i * _RB + r
        s = starts_ref[n]
        sa = pl.multiple_of((s // 128) * 128, 128)
        off = s - sa
        c = jnp.minimum(counts_ref[n], mne_ref[0])
        w = val_ref[pl.ds(0, 1), pl.ds(sa, 256)]
        w = pltpu.roll(w, 256 - off, 1)[:, :_M]
        out_ref[pl.ds(r, 1), :] = jnp.where(iot < c, w, _NEG)


def _mailbox(starts, counts, mne, val_row):
    return pl.pallas_call(
        _mailbox_body,
        grid=(_N // _RB,),
        in_specs=[
            pl.BlockSpec(memory_space=pltpu.SMEM),
            pl.BlockSpec(memory_space=pltpu.SMEM),
            pl.BlockSpec(memory_space=pltpu.SMEM),
            pl.BlockSpec((1, _E + 256), lambda i: (0, 0)),
        ],
        out_specs=pl.BlockSpec((_RB, _M), lambda i: (i, 0)),
        out_shape=jax.ShapeDtypeStruct((_N, _M), jnp.float32),
    )(starts, counts, mne, val_row)


def kernel(node_feature, W1, b1, W2, b2, src_idx, dst_idx, maximum_num_enemy,
           attack_edge_type_index):
    nf = node_feature.astype(jnp.float32)
    dst = dst_idx.astype(jnp.int32)
    src = src_idx.astype(jnp.int32)

    w1a = W1[:_D]
    w1b = W1[_D:]
    b1r = b1.reshape(1, _H)
    w2_row = W2.reshape(1, _H)
    b2r = b2.reshape(1, 1)

    a_tab, b_tab = _node_tables(nf, w1a, w1b, b1r)
    ng = 5
    eg = _E // ng
    vals = []
    for g in range(ng):
        sl = slice(g * eg, (g + 1) * eg)
        z_g = _gather_add(a_tab, b_tab, dst[sl], src[sl])
        vals.append(_mish_dot(z_g, w2_row, b2r))
    val = jnp.concatenate(vals)

    counts = jnp.bincount(dst, length=_N).astype(jnp.int32)
    starts = (jnp.cumsum(counts) - counts).astype(jnp.int32)
    val_row = jnp.concatenate([val, jnp.zeros((256,), jnp.float32)]).reshape(1, _E + 256)
    mne = jnp.asarray(maximum_num_enemy, jnp.int32).reshape(1)

    return _mailbox(starts, counts, mne, val_row)
